# baseline (speedup 1.0000x reference)
import numpy as np
import jax
import jax.numpy as jnp
from jax import lax
from jax.experimental import pallas as pl
from jax.experimental.pallas import tpu as pltpu
from jax.experimental.pallas import tpu_sc as plsc

_VOCAB = 100000
_D = 64
_B = 4096
_L = 200
_SCALE = 8.0

_NC, _NS, _LANES = 2, 16, 16
_NW = _NC * _NS
_ROWS = _B * _L
_RPW = _ROWS // _NW
_CHUNK = 128
_K = 1
_PCHUNK = _K * _CHUNK
_NPHASE = _RPW // _PCHUNK
_NCHUNK = _RPW // _CHUNK
_VPR = _D // _LANES
_PE_VREGS = _L * _VPR * -(-(_L * _VPR + _PCHUNK * _VPR) // (_L * _VPR))
_PE_REP = _PE_VREGS // (_L * _VPR)


def _pos_encoding():
    depth_per_part = _D // 2
    positions = np.arange(_L)[:, np.newaxis]
    rates = np.arange(depth_per_part)[np.newaxis, :]
    angle_rates = 1 / np.power(10000, 2 * rates / np.float32(_D))
    rads = positions * angle_rates
    return np.concatenate([np.sin(rads), np.cos(rads)], axis=-1).astype(np.float32)


def _body(x_hbm, pe2_hbm, table_hbm, out_hbm, idx_v, pe2_v, rows_v, sidx_v,
          obuf_v, gsem0, gsem1, osem0, osem1):
    wid = lax.axis_index("s") * _NC + lax.axis_index("c")
    base = wid * _RPW
    pltpu.sync_copy(x_hbm.at[wid], idx_v)
    pltpu.sync_copy(pe2_hbm, pe2_v)

    gsems = (gsem0, gsem1)
    osems = (osem0, osem1)

    def halve(s, b):
        for k in range(_K):
            for t in range(_CHUNK // _LANES):
                sl = pl.ds(t * _LANES, _LANES)
                sidx_v[b, k, sl] = lax.shift_right_logical(
                    idx_v[s * _K + k, sl], 1)

    def fire_gather(s, b):
        for k in range(_K):
            pltpu.async_copy(
                table_hbm.at[sidx_v.at[b, k]],
                rows_v.at[b, pl.ds(k * _CHUNK, _CHUNK)], gsems[b])

    def drain_gather(s, b):
        for k in range(_K):
            pltpu.make_async_copy(
                table_hbm.at[sidx_v.at[b, k]],
                rows_v.at[b, pl.ds(k * _CHUNK, _CHUNK)], gsems[b]).wait()

    def fire_out(s, b):
        pltpu.async_copy(
            obuf_v.at[b], out_hbm.at[pl.ds(base + s * _PCHUNK, _PCHUNK)],
            osems[b])

    def drain_out(s, b):
        pltpu.make_async_copy(
            obuf_v.at[b], out_hbm.at[pl.ds(base + s * _PCHUNK, _PCHUNK)],
            osems[b]).wait()

    def compute(s, b):
        buf = rows_v.at[b]
        ob = obuf_v.at[b]
        p0 = lax.rem(s * (_PCHUNK * _VPR), _L * _VPR)

        def sub_fn(st, carry):
            row2d = s * _K + lax.shift_right_logical(st, 3)
            col0 = lax.rem(st, 8) * _LANES
            parvec = lax.rem(idx_v[row2d, pl.ds(col0, _LANES)], 2
                             ).astype(jnp.float32)
            pst = p0 + st * (_LANES * _VPR)
            for j in range(_LANES):
                par = jnp.take(parvec, jnp.full((_LANES,), j, jnp.int32))
                i = st * _LANES + j
                for t in range(_VPR):
                    sl = pl.ds(t * _LANES, _LANES)
                    hi = buf[i, pl.ds(_D + t * _LANES, _LANES)]
                    lo = buf[i, sl]
                    ob[i, sl] = ((lo + par * (hi - lo)) * _SCALE
                                 + pe2_v[pst + _VPR * j + t])
            return carry

        lax.fori_loop(0, _PCHUNK // _LANES, sub_fn, 0)

    def phase(s, b, first, fire_next):
        drain_gather(s, b)
        if not first:
            drain_out(s - 1, 1 - b)
        if fire_next:
            halve(s + 1, 1 - b)
            fire_gather(s + 1, 1 - b)
        compute(s, b)
        fire_out(s, b)

    halve(0, 0)
    fire_gather(0, 0)
    phase(0, 0, True, True)
    phase(1, 1, False, True)

    def pair(k, carry):
        j = 2 + 2 * k
        phase(j, 0, False, True)
        phase(j + 1, 1, False, True)
        return carry

    lax.fori_loop(0, (_NPHASE - 4) // 2, pair, 0)

    phase(_NPHASE - 2, 0, False, True)
    phase(_NPHASE - 1, 1, False, False)
    drain_out(_NPHASE - 1, 1)


def kernel(x, table):
    pe = _pos_encoding()
    pe2 = np.concatenate([pe] * _PE_REP, axis=0).reshape(_PE_VREGS, _LANES)
    xr = x.reshape(_NW, _NCHUNK, _CHUNK)
    mesh = plsc.VectorSubcoreMesh(
        core_axis_name="c", subcore_axis_name="s",
        num_cores=_NC, num_subcores=_NS)
    out = pl.kernel(
        _body,
        out_type=jax.ShapeDtypeStruct((_ROWS, _D), jnp.float32),
        mesh=mesh,
        compiler_params=pltpu.CompilerParams(use_tc_tiling_on_sc=False),
        scratch_types=[
            pltpu.VMEM((_NCHUNK, _CHUNK), jnp.int32),
            pltpu.VMEM((_PE_VREGS, _LANES), jnp.float32),
            pltpu.VMEM((2, _PCHUNK, 128), jnp.float32),
            pltpu.VMEM((2, _K, _CHUNK), jnp.int32),
            pltpu.VMEM((2, _PCHUNK, _D), jnp.float32),
            pltpu.SemaphoreType.DMA,
            pltpu.SemaphoreType.DMA,
            pltpu.SemaphoreType.DMA,
            pltpu.SemaphoreType.DMA,
        ],
    )(xr, jnp.asarray(pe2), table.reshape(_VOCAB // 2, 2 * _D))
    return out.reshape(_B, _L, _D)


# speedup vs baseline: 1.8166x; 1.8166x over previous
import numpy as np
import jax
import jax.numpy as jnp
from jax import lax
from jax.experimental import pallas as pl
from jax.experimental.pallas import tpu as pltpu
from jax.experimental.pallas import tpu_sc as plsc

_VOCAB = 100000
_D = 64
_B = 4096
_L = 200
_SCALE = 8.0

_NC, _NS, _LANES = 2, 16, 16
_NW = _NC * _NS
_ROWS = _B * _L
_RPW = _ROWS // _NW
_CHUNK = 128
_K = 2
_PCHUNK = _K * _CHUNK
_NPHASE = _RPW // _PCHUNK
_NCHUNK = _RPW // _CHUNK
_VPR = _D // _LANES
_PE_VREGS = _L * _VPR * -(-(_L * _VPR + _PCHUNK * _VPR) // (_L * _VPR))
_PE_REP = _PE_VREGS // (_L * _VPR)


def _pos_encoding():
    depth_per_part = _D // 2
    positions = np.arange(_L)[:, np.newaxis]
    rates = np.arange(depth_per_part)[np.newaxis, :]
    angle_rates = 1 / np.power(10000, 2 * rates / np.float32(_D))
    rads = positions * angle_rates
    return np.concatenate([np.sin(rads), np.cos(rads)], axis=-1).astype(np.float32)


def _body(x_hbm, pe2_hbm, table_hbm, out_hbm, idx_v, pe2_v, rows_v, sidx_v,
          gsem0, gsem1, osem0, osem1):
    wid = lax.axis_index("s") * _NC + lax.axis_index("c")
    base = wid * _RPW
    pltpu.sync_copy(x_hbm.at[wid], idx_v)
    pltpu.sync_copy(pe2_hbm, pe2_v)

    gsems = (gsem0, gsem1)
    osems = (osem0, osem1)

    def halve(s, b):
        for k in range(_K):
            for t in range(_CHUNK // _LANES):
                sl = pl.ds(t * _LANES, _LANES)
                sidx_v[b, k, sl] = lax.shift_right_logical(
                    idx_v[s * _K + k, sl], 1)

    def fire_gather(s, b):
        for k in range(_K):
            pltpu.async_copy(
                table_hbm.at[sidx_v.at[b, k]],
                rows_v.at[b, pl.ds(k * _CHUNK, _CHUNK)], gsems[b])

    def drain_gather(s, b):
        for k in range(_K):
            pltpu.make_async_copy(
                table_hbm.at[sidx_v.at[b, k]],
                rows_v.at[b, pl.ds(k * _CHUNK, _CHUNK)], gsems[b]).wait()

    def fire_out(s, b):
        pltpu.async_copy(
            rows_v.at[b], out_hbm.at[pl.ds(base + s * _PCHUNK, _PCHUNK)],
            osems[b])

    def drain_out(s, b):
        pltpu.make_async_copy(
            rows_v.at[b], out_hbm.at[pl.ds(base + s * _PCHUNK, _PCHUNK)],
            osems[b]).wait()

    def compute(s, b):
        buf = rows_v.at[b]
        p0 = lax.rem(s * (_PCHUNK * _VPR), _L * _VPR)

        def sub_fn(st, carry):
            row2d = s * _K + lax.shift_right_logical(st, 3)
            col0 = lax.rem(st, 8) * _LANES
            parvec = lax.rem(idx_v[row2d, pl.ds(col0, _LANES)], 2
                             ).astype(jnp.float32)
            pst = p0 + st * (_LANES * _VPR)
            for j in range(_LANES):
                par = jnp.take(parvec, jnp.full((_LANES,), j, jnp.int32))
                i = st * _LANES + j
                for t in range(_VPR):
                    sl = pl.ds(t * _LANES, _LANES)
                    hi = buf[i, pl.ds(_D + t * _LANES, _LANES)]
                    lo = buf[i, sl]
                    buf[i, sl] = ((lo + par * (hi - lo)) * _SCALE
                                  + pe2_v[pst + _VPR * j + t])
            return carry

        lax.fori_loop(0, _PCHUNK // _LANES, sub_fn, 0)

    def phase(s, b, first, fire_next):
        drain_gather(s, b)
        if not first:
            drain_out(s - 1, 1 - b)
        if fire_next:
            halve(s + 1, 1 - b)
            fire_gather(s + 1, 1 - b)
        compute(s, b)
        fire_out(s, b)

    halve(0, 0)
    fire_gather(0, 0)
    phase(0, 0, True, True)
    phase(1, 1, False, True)

    def pair(k, carry):
        j = 2 + 2 * k
        phase(j, 0, False, True)
        phase(j + 1, 1, False, True)
        return carry

    lax.fori_loop(0, (_NPHASE - 4) // 2, pair, 0)

    phase(_NPHASE - 2, 0, False, True)
    phase(_NPHASE - 1, 1, False, False)
    drain_out(_NPHASE - 1, 1)


def kernel(x, table):
    pe = _pos_encoding()
    pe2 = np.concatenate([pe] * _PE_REP, axis=0).reshape(_PE_VREGS, _LANES)
    xr = x.reshape(_NW, _NCHUNK, _CHUNK)
    mesh = plsc.VectorSubcoreMesh(
        core_axis_name="c", subcore_axis_name="s",
        num_cores=_NC, num_subcores=_NS)
    out = pl.kernel(
        _body,
        out_type=jax.ShapeDtypeStruct((_ROWS, 128), jnp.float32),
        mesh=mesh,
        compiler_params=pltpu.CompilerParams(use_tc_tiling_on_sc=False),
        scratch_types=[
            pltpu.VMEM((_NCHUNK, _CHUNK), jnp.int32),
            pltpu.VMEM((_PE_VREGS, _LANES), jnp.float32),
            pltpu.VMEM((2, _PCHUNK, 128), jnp.float32),
            pltpu.VMEM((2, _K, _CHUNK), jnp.int32),
            pltpu.SemaphoreType.DMA,
            pltpu.SemaphoreType.DMA,
            pltpu.SemaphoreType.DMA,
            pltpu.SemaphoreType.DMA,
        ],
    )(xr, jnp.asarray(pe2), table.reshape(_VOCAB // 2, 2 * _D))
    return out[:, :_D].reshape(_B, _L, _D)
